# baseline (speedup 1.0000x reference)
import jax
import jax.numpy as jnp
from jax import lax
from jax.experimental import pallas as pl
from jax.experimental.pallas import tpu as pltpu
from jax.experimental.pallas import tpu_sc as plsc

B = 4096
L = 50
IN = 128
V = 100000
D = IN + 1

NC = 2
NS = 16
LANES = 16
NW = NC * NS
BW = B // NW
RPC = 2
CIDS = RPC * L
CPAD = 104
NCH = BW // RPC
LP = 64


def _sc_body(x_hbm, ids_hbm, emb_hbm, bias_hbm, out_hbm,
             ids_v, x_v, rows0, rows1, bvec0, bvec1, logits_v, csums_v,
             sem0, sem1):
    wid = lax.axis_index("s") * NC + lax.axis_index("c")
    base = wid * BW

    pltpu.sync_copy(ids_hbm.at[wid], ids_v)
    pltpu.sync_copy(x_hbm.at[pl.ds(base, BW)], x_v)

    def idx_ref(c):
        return ids_v.at[pl.ds(pl.multiple_of(c * CPAD, 8), CIDS)]

    def start_gather(c, buf, bvec, sem):
        pltpu.async_copy(emb_hbm.at[idx_ref(c), pl.ds(0, IN)], buf, sem)
        pltpu.async_copy(bias_hbm.at[idx_ref(c)], bvec, sem)

    def wait_gather(c, buf, bvec, sem):
        pltpu.make_async_copy(
            emb_hbm.at[idx_ref(c), pl.ds(0, IN)], buf, sem).wait()
        pltpu.make_async_copy(bias_hbm.at[idx_ref(c)], bvec, sem).wait()

    start_gather(0, rows0, bvec0, sem0)
    start_gather(1, rows1, bvec1, sem1)

    lane = lax.iota(jnp.int32, LANES)
    g_row = [lane + g * LANES for g in range(L // LANES + 1)]
    col15 = jnp.full((LANES,), LANES - 1, jnp.int32)
    b_row = [[jnp.minimum(g * LANES + lane, L - 1) + br * L
              for g in range(L // LANES + 1)] for br in range(RPC)]

    def compute(c, buf, bvec):
        for br in range(RPC):
            b2 = c * RPC + br
            rbase = br * L
            xc = [x_v[b2, pl.ds(16 * k, LANES)] for k in range(IN // LANES)]

            for l in range(L):
                r = rbase + l
                acc = xc[0] * buf[r, pl.ds(0, LANES)]
                for k in range(1, IN // LANES):
                    acc = acc + xc[k] * buf[r, pl.ds(16 * k, LANES)]
                csums_v[l, pl.ds(0, LANES)] = plsc.cumsum(acc)

            for g in range(L // LANES + 1):
                dots = plsc.load_gather(csums_v, [g_row[g], col15])
                bias = plsc.load_gather(bvec, [b_row[br][g]])
                logits_v[b2, pl.ds(g * LANES, LANES)] = dots + bias

    def half(c, buf, bvec, sem):
        wait_gather(c, buf, bvec, sem)
        compute(c, buf, bvec)

        @pl.when(c + 2 < NCH)
        def _():
            start_gather(c + 2, buf, bvec, sem)

    @pl.loop(0, NCH, step=2)
    def _(c):
        half(c, rows0, bvec0, sem0)
        half(c + 1, rows1, bvec1, sem1)

    pltpu.sync_copy(logits_v, out_hbm.at[pl.ds(base, BW)])


def _tc_body(lg_ref, tv_ref, tm_ref, loss_ref, sig_ref):
    lg = lg_ref[:, :L]
    tv = tv_ref[...]
    tm = tm_ref[...]
    elem = jnp.maximum(lg, 0.0) - lg * tv + jnp.log1p(jnp.exp(-jnp.abs(lg)))
    loss_ref[0, 0] = jnp.sum(tm * elem) / (B * L)
    sig_ref[...] = jax.nn.sigmoid(lg)


def kernel(x, target_ids, target_values, target_mask, emb_weight):
    ids3 = target_ids.astype(jnp.int32).reshape(NW, NCH, CIDS)
    ids2 = jnp.pad(ids3, ((0, 0), (0, 0), (0, CPAD - CIDS))).reshape(
        NW, NCH * CPAD)
    bias_col = emb_weight[:, IN]

    mesh = plsc.VectorSubcoreMesh(core_axis_name="c", subcore_axis_name="s")
    logits_full = pl.kernel(
        _sc_body,
        out_type=jax.ShapeDtypeStruct((B, 128), jnp.float32),
        mesh=mesh,
        scratch_types=[
            pltpu.VMEM((NCH * CPAD,), jnp.int32),
            pltpu.VMEM((BW, IN), jnp.float32),
            pltpu.VMEM((CIDS, IN), jnp.float32),
            pltpu.VMEM((CIDS, IN), jnp.float32),
            pltpu.VMEM((CIDS,), jnp.float32),
            pltpu.VMEM((CIDS,), jnp.float32),
            pltpu.VMEM((BW, 128), jnp.float32),
            pltpu.VMEM((LP, LANES), jnp.float32),
            pltpu.SemaphoreType.DMA,
            pltpu.SemaphoreType.DMA,
        ],
        compiler_params=pltpu.CompilerParams(needs_layout_passes=False),
    )(x, ids2, emb_weight, bias_col)

    loss2d, sig = pl.pallas_call(
        _tc_body,
        out_shape=(
            jax.ShapeDtypeStruct((1, 1), jnp.float32),
            jax.ShapeDtypeStruct((B, L), jnp.float32),
        ),
        out_specs=(
            pl.BlockSpec(memory_space=pltpu.SMEM),
            pl.BlockSpec(),
        ),
    )(logits_full, target_values, target_mask)

    return (loss2d[0, 0], sig)


# speedup vs baseline: 1.0180x; 1.0180x over previous
import jax
import jax.numpy as jnp
from jax import lax
from jax.experimental import pallas as pl
from jax.experimental.pallas import tpu as pltpu
from jax.experimental.pallas import tpu_sc as plsc

B = 4096
L = 50
IN = 128
V = 100000
D = IN + 1

NC = 2
NS = 16
LANES = 16
NW = NC * NS
BW = B // NW
RPC = 2
CIDS = RPC * L
CPAD = 104
NCH = BW // RPC
LP = 64


def _sc_body(x_hbm, ids_hbm, emb_hbm, bias_hbm, out_hbm,
             ids_v, x_v, rows0, rows1, bvec0, bvec1, logits_v, csums_v,
             sem0, sem1):
    wid = lax.axis_index("s") * NC + lax.axis_index("c")
    base = wid * BW

    pltpu.sync_copy(ids_hbm.at[wid], ids_v)
    pltpu.sync_copy(x_hbm.at[pl.ds(base, BW)], x_v)

    def idx_ref(c):
        return ids_v.at[pl.ds(pl.multiple_of(c * CPAD, 8), CIDS)]

    def start_gather(c, buf, bvec, sem):
        pltpu.async_copy(emb_hbm.at[idx_ref(c), pl.ds(0, IN)], buf, sem)
        pltpu.async_copy(bias_hbm.at[idx_ref(c)], bvec, sem)

    def wait_gather(c, buf, bvec, sem):
        pltpu.make_async_copy(
            emb_hbm.at[idx_ref(c), pl.ds(0, IN)], buf, sem).wait()
        pltpu.make_async_copy(bias_hbm.at[idx_ref(c)], bvec, sem).wait()

    start_gather(0, rows0, bvec0, sem0)
    start_gather(1, rows1, bvec1, sem1)

    lane = lax.iota(jnp.int32, LANES)
    g_row = [lane + g * LANES for g in range(L // LANES + 1)]
    col15 = jnp.full((LANES,), LANES - 1, jnp.int32)
    b_row = [[jnp.minimum(g * LANES + lane, L - 1) + br * L
              for g in range(L // LANES + 1)] for br in range(RPC)]

    def compute(c, buf, bvec):
        for br in range(RPC):
            b2 = c * RPC + br
            rbase = br * L
            xc = [x_v[b2, pl.ds(16 * k, LANES)] for k in range(IN // LANES)]

            for l in range(0, L, 2):
                r0 = rbase + l
                r1 = rbase + l + 1
                acc0 = xc[0] * buf[r0, pl.ds(0, LANES)]
                acc1 = xc[0] * buf[r1, pl.ds(0, LANES)]
                for k in range(1, IN // LANES):
                    acc0 = acc0 + xc[k] * buf[r0, pl.ds(16 * k, LANES)]
                    acc1 = acc1 + xc[k] * buf[r1, pl.ds(16 * k, LANES)]
                csums_v[l, pl.ds(0, LANES)] = plsc.cumsum(acc0)
                csums_v[l + 1, pl.ds(0, LANES)] = plsc.cumsum(acc1)

            for g in range(L // LANES + 1):
                dots = plsc.load_gather(csums_v, [g_row[g], col15])
                bias = plsc.load_gather(bvec, [b_row[br][g]])
                logits_v[b2, pl.ds(g * LANES, LANES)] = dots + bias

    def half(c, buf, bvec, sem):
        wait_gather(c, buf, bvec, sem)
        compute(c, buf, bvec)

        @pl.when(c + 2 < NCH)
        def _():
            start_gather(c + 2, buf, bvec, sem)

    @pl.loop(0, NCH, step=2)
    def _(c):
        half(c, rows0, bvec0, sem0)
        half(c + 1, rows1, bvec1, sem1)

    pltpu.sync_copy(logits_v, out_hbm.at[pl.ds(base, BW)])


def _tc_body(lg_ref, tv_ref, tm_ref, loss_ref, sig_ref):
    lg = lg_ref[:, :L]
    tv = tv_ref[...]
    tm = tm_ref[...]
    elem = jnp.maximum(lg, 0.0) - lg * tv + jnp.log1p(jnp.exp(-jnp.abs(lg)))
    loss_ref[0, 0] = jnp.sum(tm * elem) / (B * L)
    sig_ref[...] = jax.nn.sigmoid(lg)


def kernel(x, target_ids, target_values, target_mask, emb_weight):
    ids3 = target_ids.astype(jnp.int32).reshape(NW, NCH, CIDS)
    ids2 = jnp.pad(ids3, ((0, 0), (0, 0), (0, CPAD - CIDS))).reshape(
        NW, NCH * CPAD)
    bias_col = emb_weight[:, IN]

    mesh = plsc.VectorSubcoreMesh(core_axis_name="c", subcore_axis_name="s")
    logits_full = pl.kernel(
        _sc_body,
        out_type=jax.ShapeDtypeStruct((B, 128), jnp.float32),
        mesh=mesh,
        scratch_types=[
            pltpu.VMEM((NCH * CPAD,), jnp.int32),
            pltpu.VMEM((BW, IN), jnp.float32),
            pltpu.VMEM((CIDS, IN), jnp.float32),
            pltpu.VMEM((CIDS, IN), jnp.float32),
            pltpu.VMEM((CIDS,), jnp.float32),
            pltpu.VMEM((CIDS,), jnp.float32),
            pltpu.VMEM((BW, 128), jnp.float32),
            pltpu.VMEM((LP, LANES), jnp.float32),
            pltpu.SemaphoreType.DMA,
            pltpu.SemaphoreType.DMA,
        ],
        compiler_params=pltpu.CompilerParams(needs_layout_passes=False),
    )(x, ids2, emb_weight, bias_col)

    loss2d, sig = pl.pallas_call(
        _tc_body,
        out_shape=(
            jax.ShapeDtypeStruct((1, 1), jnp.float32),
            jax.ShapeDtypeStruct((B, L), jnp.float32),
        ),
        out_specs=(
            pl.BlockSpec(memory_space=pltpu.SMEM),
            pl.BlockSpec(),
        ),
    )(logits_full, target_values, target_mask)

    return (loss2d[0, 0], sig)


# speedup vs baseline: 1.1209x; 1.1011x over previous
import jax
import jax.numpy as jnp
from jax import lax
from jax.experimental import pallas as pl
from jax.experimental.pallas import tpu as pltpu
from jax.experimental.pallas import tpu_sc as plsc

B = 4096
L = 50
IN = 128
V = 100000
D = IN + 1

NC = 2
NS = 16
LANES = 16
NW = NC * NS
BW = B // NW
RPC = 2
CIDS = RPC * L
CPAD = 104
NCH = BW // RPC
LP = 64


def _sc_body(x_hbm, ids_hbm, emb_hbm, bias_hbm, out_hbm,
             ids_v, x_v, rows0, rows1, bvec0, bvec1, logits_v, csums_v,
             sem0, sem1):
    wid = lax.axis_index("s") * NC + lax.axis_index("c")
    base = wid * BW

    pltpu.sync_copy(ids_hbm.at[wid], ids_v)
    pltpu.sync_copy(x_hbm.at[pl.ds(base, BW)], x_v)

    def idx_ref(c):
        return ids_v.at[pl.ds(pl.multiple_of(c * CPAD, 8), CIDS)]

    def start_gather(c, buf, bvec, sem):
        pltpu.async_copy(emb_hbm.at[idx_ref(c), pl.ds(0, IN)], buf, sem)
        pltpu.async_copy(bias_hbm.at[idx_ref(c)], bvec, sem)

    def wait_gather(c, buf, bvec, sem):
        pltpu.make_async_copy(
            emb_hbm.at[idx_ref(c), pl.ds(0, IN)], buf, sem).wait()
        pltpu.make_async_copy(bias_hbm.at[idx_ref(c)], bvec, sem).wait()

    start_gather(0, rows0, bvec0, sem0)
    start_gather(1, rows1, bvec1, sem1)

    lane = lax.iota(jnp.int32, LANES)
    g_row = [lane + g * LANES for g in range(L // LANES + 1)]
    col15 = jnp.full((LANES,), LANES - 1, jnp.int32)
    b_row = [[jnp.minimum(g * LANES + lane, L - 1) + br * L
              for g in range(L // LANES + 1)] for br in range(RPC)]

    def compute(c, buf, bvec):
        for br in range(RPC):
            b2 = c * RPC + br
            rbase = br * L
            xc = [x_v[b2, pl.ds(16 * k, LANES)] for k in range(IN // LANES)]

            for l in range(0, L, 2):
                r0 = rbase + l
                r1 = rbase + l + 1
                acc0 = xc[0] * buf[r0, pl.ds(0, LANES)]
                acc1 = xc[0] * buf[r1, pl.ds(0, LANES)]
                for k in range(1, IN // LANES):
                    acc0 = acc0 + xc[k] * buf[r0, pl.ds(16 * k, LANES)]
                    acc1 = acc1 + xc[k] * buf[r1, pl.ds(16 * k, LANES)]
                csums_v[l, pl.ds(0, LANES)] = plsc.cumsum(acc0)
                csums_v[l + 1, pl.ds(0, LANES)] = plsc.cumsum(acc1)

            for g in range(L // LANES + 1):
                dots = plsc.load_gather(csums_v, [g_row[g], col15])
                bias = plsc.load_gather(bvec, [b_row[br][g]])
                logits_v[b2, pl.ds(g * LANES, LANES)] = dots + bias

    def half(c, buf, bvec, sem):
        wait_gather(c, buf, bvec, sem)
        compute(c, buf, bvec)

        @pl.when(c + 2 < NCH)
        def _():
            start_gather(c + 2, buf, bvec, sem)

    @pl.loop(0, NCH, step=2)
    def _(c):
        half(c, rows0, bvec0, sem0)
        half(c + 1, rows1, bvec1, sem1)

    pltpu.sync_copy(logits_v, out_hbm.at[pl.ds(base, BW)])


def _tc_body(lg_ref, tv_ref, tm_ref, loss_ref, sig_ref):
    lg = lg_ref[:, :L]
    tv = tv_ref[...]
    tm = tm_ref[...]
    elem = jnp.maximum(lg, 0.0) - lg * tv + jnp.log1p(jnp.exp(-jnp.abs(lg)))
    loss_ref[0, 0] = jnp.sum(tm * elem) / (B * L)
    sig_ref[...] = jax.nn.sigmoid(lg)


def kernel(x, target_ids, target_values, target_mask, emb_weight):
    ids3 = target_ids.astype(jnp.int32).reshape(NW, NCH, CIDS)
    ids2 = jnp.pad(ids3, ((0, 0), (0, 0), (0, CPAD - CIDS))).reshape(
        NW, NCH * CPAD)
    emb128 = emb_weight[:, :IN]
    bias_col = emb_weight[:, IN]

    mesh = plsc.VectorSubcoreMesh(core_axis_name="c", subcore_axis_name="s")
    logits_full = pl.kernel(
        _sc_body,
        out_type=jax.ShapeDtypeStruct((B, 128), jnp.float32),
        mesh=mesh,
        scratch_types=[
            pltpu.VMEM((NCH * CPAD,), jnp.int32),
            pltpu.VMEM((BW, IN), jnp.float32),
            pltpu.VMEM((CIDS, IN), jnp.float32),
            pltpu.VMEM((CIDS, IN), jnp.float32),
            pltpu.VMEM((CIDS,), jnp.float32),
            pltpu.VMEM((CIDS,), jnp.float32),
            pltpu.VMEM((BW, 128), jnp.float32),
            pltpu.VMEM((LP, LANES), jnp.float32),
            pltpu.SemaphoreType.DMA,
            pltpu.SemaphoreType.DMA,
        ],
        compiler_params=pltpu.CompilerParams(needs_layout_passes=False),
    )(x, ids2, emb128, bias_col)

    loss2d, sig = pl.pallas_call(
        _tc_body,
        out_shape=(
            jax.ShapeDtypeStruct((1, 1), jnp.float32),
            jax.ShapeDtypeStruct((B, L), jnp.float32),
        ),
        out_specs=(
            pl.BlockSpec(memory_space=pltpu.SMEM),
            pl.BlockSpec(),
        ),
    )(logits_full, target_values, target_mask)

    return (loss2d[0, 0], sig)
